# baseline (speedup 1.0000x reference)
import functools

import jax
import jax.numpy as jnp
from jax import lax
from jax.experimental import pallas as pl
from jax.experimental.pallas import tpu as pltpu
from jax.experimental.pallas import tpu_sc as plsc

B, T, C = 4, 2048, 1024
E = 8
TOPK = 2
N = B * T
FF = 4 * C
CAP = int(1.2 * (N // E + 1))
CAPP = 1280
SLOTS = E * CAPP
DUMP = SLOTS
TL_SIZE = SLOTS + 128

NC, NS = 2, 16
NW = NC * NS

RT = 512
NRT = N // RT


def _routing_body(x_ref, wrt_ref, brp_ref, meta_ref, cnt_ref, c0_ref, c1_ref):
    step = pl.program_id(0)

    @pl.when(step == 0)
    def _init():
        c0_ref[...] = jnp.zeros((1, 128), jnp.float32)
        c1_ref[...] = jnp.zeros((1, 128), jnp.float32)

    li = lax.broadcasted_iota(jnp.int32, (RT, 128), 1).astype(jnp.float32)
    logits = jnp.dot(x_ref[...], wrt_ref[...], preferred_element_type=jnp.float32)
    logits = logits + brp_ref[...]
    logits = jnp.where(li < E, logits, -1e30)
    m = jnp.max(logits, axis=1, keepdims=True)
    p = jnp.exp(logits - m)
    p = jnp.where(li < E, p, 0.0)
    p = p / jnp.sum(p, axis=1, keepdims=True)

    m0 = jnp.max(p, axis=1, keepdims=True)
    am0 = jnp.min(jnp.where(p == m0, li, 1e9), axis=1, keepdims=True)
    pm = jnp.where(li == am0, -1.0, p)
    m1 = jnp.max(pm, axis=1, keepdims=True)
    am1 = jnp.min(jnp.where(pm == m1, li, 1e9), axis=1, keepdims=True)

    s = m0 + m1 + 1e-9
    w0 = m0 / s
    w1 = m1 / s
    v0 = jnp.where(w0 > 1e-9, 1.0, 0.0)
    v1 = jnp.where(w1 > 1e-9, 1.0, 0.0)

    oh0 = jnp.where(li == am0, 1.0, 0.0) * v0
    oh1 = jnp.where(li == am1, 1.0, 0.0) * v1

    ri = lax.broadcasted_iota(jnp.int32, (RT, RT), 0)
    ci = lax.broadcasted_iota(jnp.int32, (RT, RT), 1)
    ltri = jnp.where(ci < ri, 1.0, 0.0)
    excl0 = jnp.dot(ltri, oh0, preferred_element_type=jnp.float32)
    excl1 = jnp.dot(ltri, oh1, preferred_element_type=jnp.float32)

    r0 = jnp.sum(oh0 * (excl0 + c0_ref[...]), axis=1, keepdims=True)
    r1 = jnp.sum(oh1 * (excl1 + c1_ref[...]), axis=1, keepdims=True)
    c0_ref[...] = c0_ref[...] + jnp.sum(oh0, axis=0, keepdims=True)
    c1_ref[...] = c1_ref[...] + jnp.sum(oh1, axis=0, keepdims=True)

    meta = (
        jnp.where(li == 0, am0, 0.0)
        + jnp.where(li == 1, am1, 0.0)
        + jnp.where(li == 2, w0, 0.0)
        + jnp.where(li == 3, w1, 0.0)
        + jnp.where(li == 4, r0, 0.0)
        + jnp.where(li == 5, r1, 0.0)
        + jnp.where(li == 6, v0, 0.0)
        + jnp.where(li == 7, v1, 0.0)
    )
    meta_ref[...] = meta
    cnt_ref[...] = jnp.broadcast_to(c0_ref[...], (8, 128))


def _routing(xf, wrt, brp, interpret=False):
    return pl.pallas_call(
        _routing_body,
        grid=(NRT,),
        in_specs=[
            pl.BlockSpec((RT, C), lambda i: (i, 0)),
            pl.BlockSpec((C, 128), lambda i: (0, 0)),
            pl.BlockSpec((1, 128), lambda i: (0, 0)),
        ],
        out_specs=[
            pl.BlockSpec((RT, 128), lambda i: (i, 0)),
            pl.BlockSpec((8, 128), lambda i: (0, 0)),
        ],
        out_shape=[
            jax.ShapeDtypeStruct((N, 128), jnp.float32),
            jax.ShapeDtypeStruct((8, 128), jnp.float32),
        ],
        scratch_shapes=[
            pltpu.VMEM((1, 128), jnp.float32),
            pltpu.VMEM((1, 128), jnp.float32),
        ],
        compiler_params=pltpu.CompilerParams(
            dimension_semantics=("arbitrary",)),
        interpret=interpret,
    )(xf, wrt, brp)


def _finalize_body(meta_ref, cnt_ref, aux_ref):
    li = lax.broadcasted_iota(jnp.int32, (RT, 128), 1).astype(jnp.float32)
    mb = meta_ref[...]
    e0 = mb[:, 0:1]
    e1 = mb[:, 1:2]
    w0 = mb[:, 2:3]
    w1 = mb[:, 3:4]
    r0 = mb[:, 4:5]
    r1 = mb[:, 5:6]
    v0 = mb[:, 6:7]
    v1 = mb[:, 7:8]

    used0_row = jnp.minimum(float(CAP), cnt_ref[0:1, :])
    oh1 = jnp.where(li == e1, 1.0, 0.0)
    used0_e1 = jnp.sum(oh1 * used0_row, axis=1, keepdims=True)

    p0 = r0
    keep0 = (v0 > 0.0) & (p0 < CAP)
    p1 = used0_e1 + r1
    keep1 = (v1 > 0.0) & (p1 < CAP)

    slot0 = e0 * CAPP + p0
    slot1 = e1 * CAPP + p1
    ss0 = jnp.where(keep0, slot0, float(DUMP))
    ss1 = jnp.where(keep1, slot1, float(DUMP))
    sg0 = jnp.where(keep0, slot0, 0.0)
    sg1 = jnp.where(keep1, slot1, 0.0)
    g0 = jnp.where(keep0, w0, 0.0)
    g1 = jnp.where(keep1, w1, 0.0)
    sw = g0 + g1

    aux = (
        jnp.where(li == 0, ss0, 0.0)
        + jnp.where(li == 1, ss1, 0.0)
        + jnp.where(li == 2, sg0, 0.0)
        + jnp.where(li == 3, sg1, 0.0)
        + jnp.where(li == 4, g0, 0.0)
        + jnp.where(li == 5, g1, 0.0)
        + jnp.where(li == 6, sw, 0.0)
    )
    aux_ref[...] = aux


def _finalize(meta, cnt, interpret=False):
    return pl.pallas_call(
        _finalize_body,
        grid=(NRT,),
        in_specs=[
            pl.BlockSpec((RT, 128), lambda i: (i, 0)),
            pl.BlockSpec((8, 128), lambda i: (0, 0)),
        ],
        out_specs=pl.BlockSpec((RT, 128), lambda i: (i, 0)),
        out_shape=jax.ShapeDtypeStruct((N, 128), jnp.float32),
        interpret=interpret,
    )(meta, cnt)


def _sc_wid():
    return lax.axis_index("s") * NC + lax.axis_index("c")


def _scatter_tl(ss0, ss1):
    mesh = plsc.VectorSubcoreMesh(core_axis_name="c", subcore_axis_name="s")

    @functools.partial(
        pl.kernel,
        mesh=mesh,
        out_type=jax.ShapeDtypeStruct((TL_SIZE,), jnp.int32),
        scratch_types=[
            pltpu.VMEM((TL_SIZE,), jnp.int32),
            pltpu.VMEM((N,), jnp.int32),
            pltpu.VMEM((N,), jnp.int32),
        ],
        compiler_params=pltpu.CompilerParams(needs_layout_passes=False),
    )
    def k(ss0_hbm, ss1_hbm, tl_hbm, tl_v, s0_v, s1_v):
        wid = _sc_wid()

        @pl.when(wid == 0)
        def _work():
            pltpu.sync_copy(ss0_hbm, s0_v)
            pltpu.sync_copy(ss1_hbm, s1_v)
            zeros16 = jnp.zeros((16,), jnp.int32)

            def _memset(i, carry):
                tl_v[pl.ds(i * 16, 16)] = zeros16
                return carry

            lax.fori_loop(0, TL_SIZE // 16, _memset, 0)
            lane = lax.iota(jnp.int32, 16)

            def _scat0(i, carry):
                idx = s0_v[pl.ds(i * 16, 16)]
                plsc.store_scatter(tl_v, [idx], lane + i * 16)
                return carry

            def _scat1(i, carry):
                idx = s1_v[pl.ds(i * 16, 16)]
                plsc.store_scatter(tl_v, [idx], lane + i * 16)
                return carry

            lax.fori_loop(0, N // 16, _scat0, 0)
            lax.fori_loop(0, N // 16, _scat1, 0)
            pltpu.sync_copy(tl_v, tl_hbm)

    return k(ss0, ss1)


def _sc_gather(table, idx, rows_per_worker, chunk=32):
    nchunk = rows_per_worker // chunk
    d = table.shape[1]
    total = idx.shape[0]
    mesh = plsc.VectorSubcoreMesh(core_axis_name="c", subcore_axis_name="s")

    @functools.partial(
        pl.kernel,
        mesh=mesh,
        out_type=jax.ShapeDtypeStruct((total, d), jnp.float32),
        scratch_types=[
            pltpu.VMEM((rows_per_worker,), jnp.int32),
            pltpu.VMEM((chunk, d), jnp.float32),
            pltpu.VMEM((chunk, d), jnp.float32),
            pltpu.SemaphoreType.DMA,
            pltpu.SemaphoreType.DMA,
        ],
        compiler_params=pltpu.CompilerParams(needs_layout_passes=False),
    )
    def k(table_hbm, idx_hbm, out_hbm, idx_v, rows0, rows1, sem0, sem1):
        wid = _sc_wid()
        base = wid * rows_per_worker
        pltpu.sync_copy(idx_hbm.at[pl.ds(base, rows_per_worker)], idx_v)
        bufs = (rows0, rows1)
        sems = (sem0, sem1)

        def _start(c):
            return pltpu.async_copy(
                table_hbm.at[idx_v.at[pl.ds(c * chunk, chunk)]],
                bufs[c % 2], sems[c % 2])

        handles = {}
        handles[0] = _start(0)
        if nchunk > 1:
            handles[1] = _start(1)
        for c in range(nchunk):
            handles[c].wait()
            pltpu.sync_copy(bufs[c % 2],
                            out_hbm.at[pl.ds(base + c * chunk, chunk)])
            if c + 2 < nchunk:
                handles[c + 2] = _start(c + 2)

    return k(table, idx)


BF = 512
NKF = FF // BF


def _ffn_body(xg_ref, w1_ref, b1_ref, w2_ref, b2_ref, ls_ref, f_ref):
    kstep = pl.program_id(1)

    @pl.when(kstep == 0)
    def _z():
        f_ref[...] = jnp.zeros_like(f_ref)

    h = jax.lax.dot_general(
        xg_ref[...], w1_ref[0],
        (((1,), (1,)), ((), ())), preferred_element_type=jnp.float32)
    h = h + b1_ref[0]
    h = 0.5 * h * (1.0 + lax.erf(h * 0.7071067811865476))
    y = jax.lax.dot_general(
        h, w2_ref[0],
        (((1,), (1,)), ((), ())), preferred_element_type=jnp.float32)
    f_ref[...] += y

    @pl.when(kstep == NKF - 1)
    def _fin():
        f_ref[...] = (f_ref[...] + b2_ref[0]) * ls_ref[0]


def _ffn(xg, w1, b1r, w2, b2r, lsr, interpret=False):
    return pl.pallas_call(
        _ffn_body,
        grid=(E, NKF),
        in_specs=[
            pl.BlockSpec((CAPP, C), lambda e, k: (e, 0)),
            pl.BlockSpec((1, BF, C), lambda e, k: (e, k, 0)),
            pl.BlockSpec((1, 1, BF), lambda e, k: (e, 0, k)),
            pl.BlockSpec((1, C, BF), lambda e, k: (e, 0, k)),
            pl.BlockSpec((1, 1, C), lambda e, k: (e, 0, 0)),
            pl.BlockSpec((1, 1, C), lambda e, k: (e, 0, 0)),
        ],
        out_specs=pl.BlockSpec((CAPP, C), lambda e, k: (e, 0)),
        out_shape=jax.ShapeDtypeStruct((SLOTS, C), jnp.float32),
        compiler_params=pltpu.CompilerParams(
            dimension_semantics=("arbitrary", "arbitrary")),
        interpret=interpret,
    )(xg, w1, b1r, w2, b2r, lsr)


def _combine_body(x_ref, y0_ref, y1_ref, aux_ref, o_ref):
    a = aux_ref[...]
    g0 = a[:, 4:5]
    g1 = a[:, 5:6]
    sw = a[:, 6:7]
    o_ref[...] = sw * x_ref[...] + g0 * y0_ref[...] + g1 * y1_ref[...]


def _combine(xf, y0, y1, aux, interpret=False):
    return pl.pallas_call(
        _combine_body,
        grid=(NRT,),
        in_specs=[
            pl.BlockSpec((RT, C), lambda i: (i, 0)),
            pl.BlockSpec((RT, C), lambda i: (i, 0)),
            pl.BlockSpec((RT, C), lambda i: (i, 0)),
            pl.BlockSpec((RT, 128), lambda i: (i, 0)),
        ],
        out_specs=pl.BlockSpec((RT, C), lambda i: (i, 0)),
        out_shape=jax.ShapeDtypeStruct((N, C), jnp.float32),
        interpret=interpret,
    )(xf, y0, y1, aux)


def kernel(x, Wr, br, W1, b1, W2, b2, layer_scale):
    xf = x.reshape(N, C)
    wrt = jnp.zeros((C, 128), jnp.float32).at[:, :E].set(Wr.T)
    brp = jnp.zeros((1, 128), jnp.float32).at[0, :E].set(br)

    meta, cnt = _routing(xf, wrt, brp)
    aux = _finalize(meta, cnt)

    ss0 = aux[:, 0].astype(jnp.int32)
    ss1 = aux[:, 1].astype(jnp.int32)
    sg0 = aux[:, 2].astype(jnp.int32)
    sg1 = aux[:, 3].astype(jnp.int32)

    tl = _scatter_tl(ss0, ss1)

    xg = _sc_gather(xf, tl[:SLOTS], SLOTS // NW)

    f = _ffn(xg, W1, b1.reshape(E, 1, FF), W2, b2.reshape(E, 1, C),
             layer_scale.reshape(E, 1, C))

    y01 = _sc_gather(f, jnp.concatenate([sg0, sg1]), 2 * N // NW)
    y0 = y01[:N]
    y1 = y01[N:]

    out = _combine(xf, y0, y1, aux)
    return out.reshape(B, T, C)


# speedup vs baseline: 1.0032x; 1.0032x over previous
import functools

import jax
import jax.numpy as jnp
from jax import lax
from jax.experimental import pallas as pl
from jax.experimental.pallas import tpu as pltpu
from jax.experimental.pallas import tpu_sc as plsc

B, T, C = 4, 2048, 1024
E = 8
TOPK = 2
N = B * T
FF = 4 * C
CAP = int(1.2 * (N // E + 1))
CAPP = 1280
SLOTS = E * CAPP
DUMP = SLOTS
TL_SIZE = SLOTS + 128

NC, NS = 2, 16
NW = NC * NS

RT = 512
NRT = N // RT


def _routing_body(x_ref, wrt_ref, brp_ref, meta_ref, cnt_ref, c0_ref, c1_ref):
    step = pl.program_id(0)

    @pl.when(step == 0)
    def _init():
        c0_ref[...] = jnp.zeros((1, 128), jnp.float32)
        c1_ref[...] = jnp.zeros((1, 128), jnp.float32)

    li = lax.broadcasted_iota(jnp.int32, (RT, 128), 1).astype(jnp.float32)
    logits = jnp.dot(x_ref[...], wrt_ref[...], preferred_element_type=jnp.float32)
    logits = logits + brp_ref[...]
    logits = jnp.where(li < E, logits, -1e30)
    m = jnp.max(logits, axis=1, keepdims=True)
    p = jnp.exp(logits - m)
    p = jnp.where(li < E, p, 0.0)
    p = p / jnp.sum(p, axis=1, keepdims=True)

    m0 = jnp.max(p, axis=1, keepdims=True)
    am0 = jnp.min(jnp.where(p == m0, li, 1e9), axis=1, keepdims=True)
    pm = jnp.where(li == am0, -1.0, p)
    m1 = jnp.max(pm, axis=1, keepdims=True)
    am1 = jnp.min(jnp.where(pm == m1, li, 1e9), axis=1, keepdims=True)

    s = m0 + m1 + 1e-9
    w0 = m0 / s
    w1 = m1 / s
    v0 = jnp.where(w0 > 1e-9, 1.0, 0.0)
    v1 = jnp.where(w1 > 1e-9, 1.0, 0.0)

    oh0 = jnp.where(li == am0, 1.0, 0.0) * v0
    oh1 = jnp.where(li == am1, 1.0, 0.0) * v1

    ri = lax.broadcasted_iota(jnp.int32, (RT, RT), 0)
    ci = lax.broadcasted_iota(jnp.int32, (RT, RT), 1)
    ltri = jnp.where(ci < ri, 1.0, 0.0)
    excl0 = jnp.dot(ltri, oh0, preferred_element_type=jnp.float32)
    excl1 = jnp.dot(ltri, oh1, preferred_element_type=jnp.float32)

    r0 = jnp.sum(oh0 * (excl0 + c0_ref[...]), axis=1, keepdims=True)
    r1 = jnp.sum(oh1 * (excl1 + c1_ref[...]), axis=1, keepdims=True)
    c0_ref[...] = c0_ref[...] + jnp.sum(oh0, axis=0, keepdims=True)
    c1_ref[...] = c1_ref[...] + jnp.sum(oh1, axis=0, keepdims=True)

    meta = (
        jnp.where(li == 0, am0, 0.0)
        + jnp.where(li == 1, am1, 0.0)
        + jnp.where(li == 2, w0, 0.0)
        + jnp.where(li == 3, w1, 0.0)
        + jnp.where(li == 4, r0, 0.0)
        + jnp.where(li == 5, r1, 0.0)
        + jnp.where(li == 6, v0, 0.0)
        + jnp.where(li == 7, v1, 0.0)
    )
    meta_ref[...] = meta
    cnt_ref[...] = jnp.broadcast_to(c0_ref[...], (8, 128))


def _routing(xf, wrt, brp, interpret=False):
    return pl.pallas_call(
        _routing_body,
        grid=(NRT,),
        in_specs=[
            pl.BlockSpec((RT, C), lambda i: (i, 0)),
            pl.BlockSpec((C, 128), lambda i: (0, 0)),
            pl.BlockSpec((1, 128), lambda i: (0, 0)),
        ],
        out_specs=[
            pl.BlockSpec((RT, 128), lambda i: (i, 0)),
            pl.BlockSpec((8, 128), lambda i: (0, 0)),
        ],
        out_shape=[
            jax.ShapeDtypeStruct((N, 128), jnp.float32),
            jax.ShapeDtypeStruct((8, 128), jnp.float32),
        ],
        scratch_shapes=[
            pltpu.VMEM((1, 128), jnp.float32),
            pltpu.VMEM((1, 128), jnp.float32),
        ],
        compiler_params=pltpu.CompilerParams(
            dimension_semantics=("arbitrary",)),
        interpret=interpret,
    )(xf, wrt, brp)


def _finalize_body(meta_ref, cnt_ref, aux_ref):
    li = lax.broadcasted_iota(jnp.int32, (RT, 128), 1).astype(jnp.float32)
    mb = meta_ref[...]
    e0 = mb[:, 0:1]
    e1 = mb[:, 1:2]
    w0 = mb[:, 2:3]
    w1 = mb[:, 3:4]
    r0 = mb[:, 4:5]
    r1 = mb[:, 5:6]
    v0 = mb[:, 6:7]
    v1 = mb[:, 7:8]

    used0_row = jnp.minimum(float(CAP), cnt_ref[0:1, :])
    oh1 = jnp.where(li == e1, 1.0, 0.0)
    used0_e1 = jnp.sum(oh1 * used0_row, axis=1, keepdims=True)

    p0 = r0
    keep0 = (v0 > 0.0) & (p0 < CAP)
    p1 = used0_e1 + r1
    keep1 = (v1 > 0.0) & (p1 < CAP)

    slot0 = e0 * CAPP + p0
    slot1 = e1 * CAPP + p1
    ss0 = jnp.where(keep0, slot0, float(DUMP))
    ss1 = jnp.where(keep1, slot1, float(DUMP))
    sg0 = jnp.where(keep0, slot0, 0.0)
    sg1 = jnp.where(keep1, slot1, 0.0)
    g0 = jnp.where(keep0, w0, 0.0)
    g1 = jnp.where(keep1, w1, 0.0)
    sw = g0 + g1

    aux = (
        jnp.where(li == 0, ss0, 0.0)
        + jnp.where(li == 1, ss1, 0.0)
        + jnp.where(li == 2, sg0, 0.0)
        + jnp.where(li == 3, sg1, 0.0)
        + jnp.where(li == 4, g0, 0.0)
        + jnp.where(li == 5, g1, 0.0)
        + jnp.where(li == 6, sw, 0.0)
    )
    aux_ref[...] = aux


def _finalize(meta, cnt, interpret=False):
    return pl.pallas_call(
        _finalize_body,
        grid=(NRT,),
        in_specs=[
            pl.BlockSpec((RT, 128), lambda i: (i, 0)),
            pl.BlockSpec((8, 128), lambda i: (0, 0)),
        ],
        out_specs=pl.BlockSpec((RT, 128), lambda i: (i, 0)),
        out_shape=jax.ShapeDtypeStruct((N, 128), jnp.float32),
        interpret=interpret,
    )(meta, cnt)


def _sc_wid():
    return lax.axis_index("s") * NC + lax.axis_index("c")


def _scatter_tl(ss0, ss1):
    mesh = plsc.VectorSubcoreMesh(core_axis_name="c", subcore_axis_name="s")

    @functools.partial(
        pl.kernel,
        mesh=mesh,
        out_type=jax.ShapeDtypeStruct((TL_SIZE,), jnp.int32),
        scratch_types=[
            pltpu.VMEM((TL_SIZE,), jnp.int32),
            pltpu.VMEM((N,), jnp.int32),
            pltpu.VMEM((N,), jnp.int32),
        ],
        compiler_params=pltpu.CompilerParams(needs_layout_passes=False),
    )
    def k(ss0_hbm, ss1_hbm, tl_hbm, tl_v, s0_v, s1_v):
        wid = _sc_wid()

        @pl.when(wid == 0)
        def _work():
            pltpu.sync_copy(ss0_hbm, s0_v)
            pltpu.sync_copy(ss1_hbm, s1_v)
            zeros16 = jnp.zeros((16,), jnp.int32)

            def _memset(i, carry):
                tl_v[pl.ds(i * 16, 16)] = zeros16
                return carry

            lax.fori_loop(0, TL_SIZE // 16, _memset, 0)
            lane = lax.iota(jnp.int32, 16)

            def _scat0(i, carry):
                idx = s0_v[pl.ds(i * 16, 16)]
                plsc.store_scatter(tl_v, [idx], lane + i * 16)
                return carry

            def _scat1(i, carry):
                idx = s1_v[pl.ds(i * 16, 16)]
                plsc.store_scatter(tl_v, [idx], lane + i * 16)
                return carry

            lax.fori_loop(0, N // 16, _scat0, 0)
            lax.fori_loop(0, N // 16, _scat1, 0)
            pltpu.sync_copy(tl_v, tl_hbm)

    return k(ss0, ss1)


def _sc_gather(table, idx, rows_per_worker, chunk=64):
    nchunk = rows_per_worker // chunk
    d = table.shape[1]
    total = idx.shape[0]
    mesh = plsc.VectorSubcoreMesh(core_axis_name="c", subcore_axis_name="s")

    @functools.partial(
        pl.kernel,
        mesh=mesh,
        out_type=jax.ShapeDtypeStruct((total, d), jnp.float32),
        scratch_types=[
            pltpu.VMEM((rows_per_worker,), jnp.int32),
            pltpu.VMEM((chunk, d), jnp.float32),
            pltpu.SemaphoreType.DMA,
        ],
        compiler_params=pltpu.CompilerParams(needs_layout_passes=False),
    )
    def k(table_hbm, idx_hbm, out_hbm, idx_v, rows_v, sem):
        wid = _sc_wid()
        base = wid * rows_per_worker
        pltpu.sync_copy(idx_hbm.at[pl.ds(base, rows_per_worker)], idx_v)
        for c in range(nchunk):
            pltpu.async_copy(
                table_hbm.at[idx_v.at[pl.ds(c * chunk, chunk)]], rows_v, sem
            ).wait()
            pltpu.sync_copy(rows_v, out_hbm.at[pl.ds(base + c * chunk, chunk)])

    return k(table, idx)


BF = 512
NKF = FF // BF


def _ffn_body(xg_ref, w1_ref, b1_ref, w2_ref, b2_ref, ls_ref, f_ref):
    kstep = pl.program_id(1)

    @pl.when(kstep == 0)
    def _z():
        f_ref[...] = jnp.zeros_like(f_ref)

    h = jax.lax.dot_general(
        xg_ref[...], w1_ref[0],
        (((1,), (1,)), ((), ())), preferred_element_type=jnp.float32)
    h = h + b1_ref[0]
    h = 0.5 * h * (1.0 + lax.erf(h * 0.7071067811865476))
    y = jax.lax.dot_general(
        h, w2_ref[0],
        (((1,), (1,)), ((), ())), preferred_element_type=jnp.float32)
    f_ref[...] += y

    @pl.when(kstep == NKF - 1)
    def _fin():
        f_ref[...] = (f_ref[...] + b2_ref[0]) * ls_ref[0]


def _ffn(xg, w1, b1r, w2, b2r, lsr, interpret=False):
    return pl.pallas_call(
        _ffn_body,
        grid=(E, NKF),
        in_specs=[
            pl.BlockSpec((CAPP, C), lambda e, k: (e, 0)),
            pl.BlockSpec((1, BF, C), lambda e, k: (e, k, 0)),
            pl.BlockSpec((1, 1, BF), lambda e, k: (e, 0, k)),
            pl.BlockSpec((1, C, BF), lambda e, k: (e, 0, k)),
            pl.BlockSpec((1, 1, C), lambda e, k: (e, 0, 0)),
            pl.BlockSpec((1, 1, C), lambda e, k: (e, 0, 0)),
        ],
        out_specs=pl.BlockSpec((CAPP, C), lambda e, k: (e, 0)),
        out_shape=jax.ShapeDtypeStruct((SLOTS, C), jnp.float32),
        compiler_params=pltpu.CompilerParams(
            dimension_semantics=("arbitrary", "arbitrary")),
        interpret=interpret,
    )(xg, w1, b1r, w2, b2r, lsr)


def _combine_body(x_ref, y0_ref, y1_ref, aux_ref, o_ref):
    a = aux_ref[...]
    g0 = a[:, 4:5]
    g1 = a[:, 5:6]
    sw = a[:, 6:7]
    o_ref[...] = sw * x_ref[...] + g0 * y0_ref[...] + g1 * y1_ref[...]


def _combine(xf, y0, y1, aux, interpret=False):
    return pl.pallas_call(
        _combine_body,
        grid=(NRT,),
        in_specs=[
            pl.BlockSpec((RT, C), lambda i: (i, 0)),
            pl.BlockSpec((RT, C), lambda i: (i, 0)),
            pl.BlockSpec((RT, C), lambda i: (i, 0)),
            pl.BlockSpec((RT, 128), lambda i: (i, 0)),
        ],
        out_specs=pl.BlockSpec((RT, C), lambda i: (i, 0)),
        out_shape=jax.ShapeDtypeStruct((N, C), jnp.float32),
        interpret=interpret,
    )(xf, y0, y1, aux)


def kernel(x, Wr, br, W1, b1, W2, b2, layer_scale):
    xf = x.reshape(N, C)
    wrt = jnp.zeros((C, 128), jnp.float32).at[:, :E].set(Wr.T)
    brp = jnp.zeros((1, 128), jnp.float32).at[0, :E].set(br)

    meta, cnt = _routing(xf, wrt, brp)
    aux = _finalize(meta, cnt)

    ss0 = aux[:, 0].astype(jnp.int32)
    ss1 = aux[:, 1].astype(jnp.int32)
    sg0 = aux[:, 2].astype(jnp.int32)
    sg1 = aux[:, 3].astype(jnp.int32)

    tl = _scatter_tl(ss0, ss1)

    xg = _sc_gather(xf, tl[:SLOTS], SLOTS // NW)

    f = _ffn(xg, W1, b1.reshape(E, 1, FF), W2, b2.reshape(E, 1, C),
             layer_scale.reshape(E, 1, C))

    y01 = _sc_gather(f, jnp.concatenate([sg0, sg1]), 2 * N // NW)
    y0 = y01[:N]
    y1 = y01[N:]

    out = _combine(xf, y0, y1, aux)
    return out.reshape(B, T, C)


# speedup vs baseline: 1.0934x; 1.0899x over previous
import functools

import jax
import jax.numpy as jnp
from jax import lax
from jax.experimental import pallas as pl
from jax.experimental.pallas import tpu as pltpu
from jax.experimental.pallas import tpu_sc as plsc

B, T, C = 4, 2048, 1024
E = 8
TOPK = 2
N = B * T
FF = 4 * C
CAP = int(1.2 * (N // E + 1))
CAPP = 1280
SLOTS = E * CAPP
DUMP = SLOTS
TL_SIZE = SLOTS + 128

NC, NS = 2, 16
NW = NC * NS

RT = 512
NRT = N // RT


def _routing_body(x_ref, wrt_ref, brp_ref, meta_ref, cnt_ref, c0_ref, c1_ref):
    step = pl.program_id(0)

    @pl.when(step == 0)
    def _init():
        c0_ref[...] = jnp.zeros((1, 128), jnp.float32)
        c1_ref[...] = jnp.zeros((1, 128), jnp.float32)

    li = lax.broadcasted_iota(jnp.int32, (RT, 128), 1).astype(jnp.float32)
    logits = jnp.dot(x_ref[...], wrt_ref[...], preferred_element_type=jnp.float32)
    logits = logits + brp_ref[...]
    logits = jnp.where(li < E, logits, -1e30)
    m = jnp.max(logits, axis=1, keepdims=True)
    p = jnp.exp(logits - m)
    p = jnp.where(li < E, p, 0.0)
    p = p / jnp.sum(p, axis=1, keepdims=True)

    m0 = jnp.max(p, axis=1, keepdims=True)
    am0 = jnp.min(jnp.where(p == m0, li, 1e9), axis=1, keepdims=True)
    pm = jnp.where(li == am0, -1.0, p)
    m1 = jnp.max(pm, axis=1, keepdims=True)
    am1 = jnp.min(jnp.where(pm == m1, li, 1e9), axis=1, keepdims=True)

    s = m0 + m1 + 1e-9
    w0 = m0 / s
    w1 = m1 / s
    v0 = jnp.where(w0 > 1e-9, 1.0, 0.0)
    v1 = jnp.where(w1 > 1e-9, 1.0, 0.0)

    oh0 = jnp.where(li == am0, 1.0, 0.0) * v0
    oh1 = jnp.where(li == am1, 1.0, 0.0) * v1

    ri = lax.broadcasted_iota(jnp.int32, (RT, RT), 0)
    ci = lax.broadcasted_iota(jnp.int32, (RT, RT), 1)
    ltri = jnp.where(ci < ri, 1.0, 0.0)
    excl0 = jnp.dot(ltri, oh0, preferred_element_type=jnp.float32)
    excl1 = jnp.dot(ltri, oh1, preferred_element_type=jnp.float32)

    r0 = jnp.sum(oh0 * (excl0 + c0_ref[...]), axis=1, keepdims=True)
    r1 = jnp.sum(oh1 * (excl1 + c1_ref[...]), axis=1, keepdims=True)
    c0_ref[...] = c0_ref[...] + jnp.sum(oh0, axis=0, keepdims=True)
    c1_ref[...] = c1_ref[...] + jnp.sum(oh1, axis=0, keepdims=True)

    meta = (
        jnp.where(li == 0, am0, 0.0)
        + jnp.where(li == 1, am1, 0.0)
        + jnp.where(li == 2, w0, 0.0)
        + jnp.where(li == 3, w1, 0.0)
        + jnp.where(li == 4, r0, 0.0)
        + jnp.where(li == 5, r1, 0.0)
        + jnp.where(li == 6, v0, 0.0)
        + jnp.where(li == 7, v1, 0.0)
    )
    meta_ref[...] = meta
    cnt_ref[...] = jnp.broadcast_to(c0_ref[...], (8, 128))


def _routing(xf, wrt, brp, interpret=False):
    return pl.pallas_call(
        _routing_body,
        grid=(NRT,),
        in_specs=[
            pl.BlockSpec((RT, C), lambda i: (i, 0)),
            pl.BlockSpec((C, 128), lambda i: (0, 0)),
            pl.BlockSpec((1, 128), lambda i: (0, 0)),
        ],
        out_specs=[
            pl.BlockSpec((RT, 128), lambda i: (i, 0)),
            pl.BlockSpec((8, 128), lambda i: (0, 0)),
        ],
        out_shape=[
            jax.ShapeDtypeStruct((N, 128), jnp.float32),
            jax.ShapeDtypeStruct((8, 128), jnp.float32),
        ],
        scratch_shapes=[
            pltpu.VMEM((1, 128), jnp.float32),
            pltpu.VMEM((1, 128), jnp.float32),
        ],
        compiler_params=pltpu.CompilerParams(
            dimension_semantics=("arbitrary",)),
        interpret=interpret,
    )(xf, wrt, brp)


def _finalize_body(meta_ref, cnt_ref, aux_ref):
    li = lax.broadcasted_iota(jnp.int32, (RT, 128), 1).astype(jnp.float32)
    mb = meta_ref[...]
    e0 = mb[:, 0:1]
    e1 = mb[:, 1:2]
    w0 = mb[:, 2:3]
    w1 = mb[:, 3:4]
    r0 = mb[:, 4:5]
    r1 = mb[:, 5:6]
    v0 = mb[:, 6:7]
    v1 = mb[:, 7:8]

    used0_row = jnp.minimum(float(CAP), cnt_ref[0:1, :])
    oh1 = jnp.where(li == e1, 1.0, 0.0)
    used0_e1 = jnp.sum(oh1 * used0_row, axis=1, keepdims=True)

    p0 = r0
    keep0 = (v0 > 0.0) & (p0 < CAP)
    p1 = used0_e1 + r1
    keep1 = (v1 > 0.0) & (p1 < CAP)

    slot0 = e0 * CAPP + p0
    slot1 = e1 * CAPP + p1
    ss0 = jnp.where(keep0, slot0, float(DUMP))
    ss1 = jnp.where(keep1, slot1, float(DUMP))
    sg0 = jnp.where(keep0, slot0, 0.0)
    sg1 = jnp.where(keep1, slot1, 0.0)
    g0 = jnp.where(keep0, w0, 0.0)
    g1 = jnp.where(keep1, w1, 0.0)
    sw = g0 + g1

    aux = (
        jnp.where(li == 0, ss0, 0.0)
        + jnp.where(li == 1, ss1, 0.0)
        + jnp.where(li == 2, sg0, 0.0)
        + jnp.where(li == 3, sg1, 0.0)
        + jnp.where(li == 4, g0, 0.0)
        + jnp.where(li == 5, g1, 0.0)
        + jnp.where(li == 6, sw, 0.0)
    )
    aux_ref[...] = aux


def _finalize(meta, cnt, interpret=False):
    return pl.pallas_call(
        _finalize_body,
        grid=(NRT,),
        in_specs=[
            pl.BlockSpec((RT, 128), lambda i: (i, 0)),
            pl.BlockSpec((8, 128), lambda i: (0, 0)),
        ],
        out_specs=pl.BlockSpec((RT, 128), lambda i: (i, 0)),
        out_shape=jax.ShapeDtypeStruct((N, 128), jnp.float32),
        interpret=interpret,
    )(meta, cnt)


def _sc_wid():
    return lax.axis_index("s") * NC + lax.axis_index("c")


def _scatter_tl(ss0, ss1):
    mesh = plsc.VectorSubcoreMesh(core_axis_name="c", subcore_axis_name="s")

    @functools.partial(
        pl.kernel,
        mesh=mesh,
        out_type=jax.ShapeDtypeStruct((TL_SIZE,), jnp.int32),
        scratch_types=[
            pltpu.VMEM((TL_SIZE,), jnp.int32),
            pltpu.VMEM((N,), jnp.int32),
            pltpu.VMEM((N,), jnp.int32),
        ],
        compiler_params=pltpu.CompilerParams(needs_layout_passes=False),
    )
    def k(ss0_hbm, ss1_hbm, tl_hbm, tl_v, s0_v, s1_v):
        wid = _sc_wid()

        @pl.when(wid == 0)
        def _work():
            pltpu.sync_copy(ss0_hbm, s0_v)
            pltpu.sync_copy(ss1_hbm, s1_v)
            zeros16 = jnp.zeros((16,), jnp.int32)

            def _memset(i, carry):
                tl_v[pl.ds(i * 16, 16)] = zeros16
                return carry

            lax.fori_loop(0, TL_SIZE // 16, _memset, 0)
            lane = lax.iota(jnp.int32, 16)

            def _scat0(i, carry):
                idx = s0_v[pl.ds(i * 16, 16)]
                plsc.store_scatter(tl_v, [idx], lane + i * 16)
                return carry

            def _scat1(i, carry):
                idx = s1_v[pl.ds(i * 16, 16)]
                plsc.store_scatter(tl_v, [idx], lane + i * 16)
                return carry

            lax.fori_loop(0, N // 16, _scat0, 0)
            lax.fori_loop(0, N // 16, _scat1, 0)
            pltpu.sync_copy(tl_v, tl_hbm)

    return k(ss0, ss1)


def _sc_gather(table, idx, rows_per_worker, chunk=64):
    nchunk = rows_per_worker // chunk
    d = table.shape[1]
    total = idx.shape[0]
    mesh = plsc.VectorSubcoreMesh(core_axis_name="c", subcore_axis_name="s")

    @functools.partial(
        pl.kernel,
        mesh=mesh,
        out_type=jax.ShapeDtypeStruct((total, d), jnp.float32),
        scratch_types=[
            pltpu.VMEM((rows_per_worker,), jnp.int32),
            pltpu.VMEM((chunk, d), jnp.float32),
            pltpu.SemaphoreType.DMA,
        ],
        compiler_params=pltpu.CompilerParams(needs_layout_passes=False),
    )
    def k(table_hbm, idx_hbm, out_hbm, idx_v, rows_v, sem):
        wid = _sc_wid()
        base = wid * rows_per_worker
        pltpu.sync_copy(idx_hbm.at[pl.ds(base, rows_per_worker)], idx_v)
        for c in range(nchunk):
            pltpu.async_copy(
                table_hbm.at[idx_v.at[pl.ds(c * chunk, chunk)]], rows_v, sem
            ).wait()
            pltpu.sync_copy(rows_v, out_hbm.at[pl.ds(base + c * chunk, chunk)])

    return k(table, idx)


BF = 512
NKF = FF // BF


def _ffn_body(xg_ref, w1_ref, b1_ref, w2_ref, b2_ref, ls_ref, f_ref):
    kstep = pl.program_id(1)

    @pl.when(kstep == 0)
    def _z():
        f_ref[...] = jnp.zeros_like(f_ref)

    h = jax.lax.dot_general(
        xg_ref[...], w1_ref[0],
        (((1,), (1,)), ((), ())), preferred_element_type=jnp.float32)
    h = h + b1_ref[0]
    h = 0.5 * h * (1.0 + lax.erf(h * 0.7071067811865476))
    y = jax.lax.dot_general(
        h, w2_ref[0],
        (((1,), (1,)), ((), ())), preferred_element_type=jnp.float32)
    f_ref[...] += y

    @pl.when(kstep == NKF - 1)
    def _fin():
        f_ref[...] = (f_ref[...] + b2_ref[0]) * ls_ref[0]


def _ffn(xg, w1, b1r, w2, b2r, lsr, interpret=False):
    return pl.pallas_call(
        _ffn_body,
        grid=(E, NKF),
        in_specs=[
            pl.BlockSpec((CAPP, C), lambda e, k: (e, 0)),
            pl.BlockSpec((1, BF, C), lambda e, k: (e, k, 0)),
            pl.BlockSpec((1, 1, BF), lambda e, k: (e, 0, k)),
            pl.BlockSpec((1, C, BF), lambda e, k: (e, 0, k)),
            pl.BlockSpec((1, 1, C), lambda e, k: (e, 0, 0)),
            pl.BlockSpec((1, 1, C), lambda e, k: (e, 0, 0)),
        ],
        out_specs=pl.BlockSpec((CAPP, C), lambda e, k: (e, 0)),
        out_shape=jax.ShapeDtypeStruct((SLOTS, C), jnp.float32),
        compiler_params=pltpu.CompilerParams(
            dimension_semantics=("arbitrary", "arbitrary")),
        interpret=interpret,
    )(xg, w1, b1r, w2, b2r, lsr)


def _combine_body(x_ref, y0_ref, y1_ref, aux_ref, o_ref):
    a = aux_ref[...]
    g0 = a[:, 4:5]
    g1 = a[:, 5:6]
    sw = a[:, 6:7]
    o_ref[...] = sw * x_ref[...] + g0 * y0_ref[...] + g1 * y1_ref[...]


def _combine(xf, y0, y1, aux, interpret=False):
    return pl.pallas_call(
        _combine_body,
        grid=(NRT,),
        in_specs=[
            pl.BlockSpec((RT, C), lambda i: (i, 0)),
            pl.BlockSpec((RT, C), lambda i: (i, 0)),
            pl.BlockSpec((RT, C), lambda i: (i, 0)),
            pl.BlockSpec((RT, 128), lambda i: (i, 0)),
        ],
        out_specs=pl.BlockSpec((RT, C), lambda i: (i, 0)),
        out_shape=jax.ShapeDtypeStruct((N, C), jnp.float32),
        interpret=interpret,
    )(xf, y0, y1, aux)


def kernel(x, Wr, br, W1, b1, W2, b2, layer_scale):
    xf = x.reshape(N, C)
    wrt = jnp.zeros((C, 128), jnp.float32).at[:, :E].set(Wr.T)
    brp = jnp.zeros((1, 128), jnp.float32).at[0, :E].set(br)

    meta, cnt = _routing(xf, wrt, brp)
    aux = _finalize(meta, cnt)

    ss0 = aux[:, 0].astype(jnp.int32)
    ss1 = aux[:, 1].astype(jnp.int32)
    sg0 = aux[:, 2].astype(jnp.int32)
    sg1 = aux[:, 3].astype(jnp.int32)

    tl = _scatter_tl(ss0, ss1)

    xg = _sc_gather(xf, tl[:SLOTS], SLOTS // NW)

    f = _ffn(xg, W1, b1.reshape(E, 1, FF), W2, b2.reshape(E, 1, C),
             layer_scale.reshape(E, 1, C))

    y0 = _sc_gather(f, sg0, N // NW)
    y1 = _sc_gather(f, sg1, N // NW)

    out = _combine(xf, y0, y1, aux)
    return out.reshape(B, T, C)
